# baseline (speedup 1.0000x reference)
import functools

import jax
import jax.numpy as jnp
from jax import lax
from jax.experimental import pallas as pl
from jax.experimental.pallas import tpu as pltpu
from jax.experimental.pallas import tpu_sc as plsc

B, A, D = 16, 2048, 256
CR = 32
POS = A // CR
NC, NS = 2, 16
NW = NC * NS
NCH = 2 * B
NSLOT = 4

_mesh = plsc.VectorSubcoreMesh(core_axis_name="c", subcore_axis_name="s")


@functools.partial(
    pl.kernel,
    mesh=_mesh,
    out_type=jax.ShapeDtypeStruct((B * A, D), jnp.float32),
    scratch_types=(
        [pltpu.VMEM((32,), jnp.int32)]
        + [pltpu.VMEM((CR, D), jnp.float32) for _ in range(3 * NSLOT + 1)]
        + [pltpu.SemaphoreType.DMA for _ in range(2 * NSLOT + 1)]
    ),
)
def _block_end_sc(atom_hbm, n_hbm, res_hbm, zero_hbm, out_hbm, n_v, *bufs):
    a_bufs = bufs[0:NSLOT]
    r_bufs = bufs[NSLOT:2 * NSLOT]
    o_bufs = bufs[2 * NSLOT:3 * NSLOT]
    z_buf = bufs[3 * NSLOT]
    in_sems = bufs[3 * NSLOT + 1: 3 * NSLOT + 1 + NSLOT]
    out_sems = bufs[3 * NSLOT + 1 + NSLOT: 3 * NSLOT + 1 + 2 * NSLOT]
    zero_sem = bufs[3 * NSLOT + 1 + 2 * NSLOT]

    w = lax.axis_index("s") * NC + lax.axis_index("c")
    pltpu.sync_copy(n_hbm, n_v)

    zero16 = jnp.zeros((16,), jnp.float32)
    del z_buf

    def info(t):
        g = t // 2
        pe = (2 * (w + g)) % POS
        p = jnp.where(t % 2 == 0, pe, POS - 1 - pe)
        n_g = n_v[pl.ds(g, 16)][0]
        vrows = n_g - p * CR
        row0 = g * A + p * CR
        return row0, vrows

    def fire_in(t, k):
        row0, vrows = info(t)

        @pl.when((t < NCH) & (vrows > 0))
        def _():
            pltpu.make_async_copy(
                atom_hbm.at[pl.ds(row0, CR)], a_bufs[k], in_sems[k]
            ).start()
            pltpu.make_async_copy(
                res_hbm.at[pl.ds(row0, CR)], r_bufs[k], in_sems[k]
            ).start()

    def process(t, k, state):
        mask, zcount = state
        row0, vrows = info(t)
        valid = vrows > 0
        pend = (mask & (1 << k)) != 0

        @pl.when(valid)
        def _():
            pltpu.make_async_copy(
                atom_hbm.at[pl.ds(row0, CR)], a_bufs[k], in_sems[k]
            ).wait()
            pltpu.make_async_copy(
                res_hbm.at[pl.ds(row0, CR)], r_bufs[k], in_sems[k]
            ).wait()

            @pl.when(pend)
            def _():
                pltpu.make_async_copy(
                    o_bufs[k], out_hbm.at[pl.ds(0, CR)], out_sems[k]
                ).wait()

            @pl.when(vrows >= CR)
            def _():
                def full_body(r, c2):
                    for c in range(D // 16):
                        sl = pl.ds(c * 16, 16)
                        o_bufs[k][r, sl] = a_bufs[k][r, sl] + r_bufs[k][r, sl]
                    return c2

                lax.fori_loop(0, CR, full_body, 0)

            @pl.when(vrows < CR)
            def _():
                def part_body(r, c2):
                    keep = r < vrows
                    for c in range(D // 16):
                        sl = pl.ds(c * 16, 16)
                        s = a_bufs[k][r, sl] + r_bufs[k][r, sl]
                        o_bufs[k][r, sl] = jnp.where(keep, s, zero16)
                    return c2

                lax.fori_loop(0, CR, part_body, 0)

            pltpu.make_async_copy(
                o_bufs[k], out_hbm.at[pl.ds(row0, CR)], out_sems[k]
            ).start()

        @pl.when(jnp.logical_not(valid))
        def _():
            pltpu.make_async_copy(
                zero_hbm, out_hbm.at[pl.ds(row0, CR)], zero_sem
            ).start()

        mask = jnp.where(valid, mask | (1 << k), mask).astype(jnp.int32)
        zcount = jnp.where(valid, zcount, zcount + 1).astype(jnp.int32)
        return mask, zcount

    fire_in(jnp.int32(0), 0)
    fire_in(jnp.int32(1), 1)
    fire_in(jnp.int32(2), 2)

    def group_body(grp, state):
        mask, zcount = state
        for kk in range(NSLOT):
            t = grp * NSLOT + kk
            fire_in(t + 3, (kk + 3) % NSLOT)
            mask, zcount = process(t, kk, (mask, zcount))
        return mask, zcount

    mask, zcount = lax.fori_loop(
        0, NCH // NSLOT, group_body, (jnp.int32(0), jnp.int32(0))
    )

    for k in range(NSLOT):
        @pl.when((mask & (1 << k)) != 0)
        def _():
            pltpu.make_async_copy(
                o_bufs[k], out_hbm.at[pl.ds(0, CR)], out_sems[k]
            ).wait()

    def zdrain(i, carry):
        pltpu.make_async_copy(
            zero_hbm, out_hbm.at[pl.ds(0, CR)], zero_sem
        ).wait()
        return carry

    lax.fori_loop(0, zcount, zdrain, 0)


def kernel(atom_features, mol_slice, residual_features):
    n = jnp.pad(mol_slice[:, 0].astype(jnp.int32), (0, 16))
    zeros_src = jnp.zeros((CR, D), jnp.float32)
    out = _block_end_sc(
        atom_features.reshape(B * A, D), n,
        residual_features.reshape(B * A, D), zeros_src,
    )
    return out.reshape(B, A, D)


# speedup vs baseline: 12.9694x; 12.9694x over previous
import functools

import jax
import jax.numpy as jnp
from jax import lax
from jax.experimental import pallas as pl
from jax.experimental.pallas import tpu as pltpu
from jax.experimental.pallas import tpu_sc as plsc

B, A, D = 16, 2048, 256
CR = 32
POS = A // CR
NC, NS = 2, 16
NW = NC * NS
NCH = 2 * B
NSLOT = 4

_mesh = plsc.VectorSubcoreMesh(core_axis_name="c", subcore_axis_name="s")


@functools.partial(
    pl.kernel,
    mesh=_mesh,
    out_type=jax.ShapeDtypeStruct((B * A, D), jnp.float32),
    scratch_types=(
        [pltpu.VMEM((32,), jnp.int32)]
        + [pltpu.VMEM((CR, D), jnp.float32) for _ in range(3 * NSLOT + 1)]
        + [pltpu.SemaphoreType.DMA for _ in range(2 * NSLOT + 1)]
    ),
)
def _block_end_sc(atom_hbm, n_hbm, res_hbm, out_hbm, n_v, *bufs):
    a_bufs = bufs[0:NSLOT]
    r_bufs = bufs[NSLOT:2 * NSLOT]
    o_bufs = bufs[2 * NSLOT:3 * NSLOT]
    z_buf = bufs[3 * NSLOT]
    in_sems = bufs[3 * NSLOT + 1: 3 * NSLOT + 1 + NSLOT]
    out_sems = bufs[3 * NSLOT + 1 + NSLOT: 3 * NSLOT + 1 + 2 * NSLOT]
    zero_sem = bufs[3 * NSLOT + 1 + 2 * NSLOT]

    w = lax.axis_index("s") * NC + lax.axis_index("c")
    pltpu.sync_copy(n_hbm, n_v)

    zero16 = jnp.zeros((16,), jnp.float32)

    def zbody(r, carry):
        for c in range(D // 16):
            z_buf[r, pl.ds(c * 16, 16)] = zero16
        return carry

    lax.fori_loop(0, CR, zbody, 0)

    def info(t):
        g = t // 2
        pe = (2 * (w + g)) % POS
        p = jnp.where(t % 2 == 0, pe, POS - 1 - pe)
        n_g = n_v[pl.ds(g, 16)][0]
        vrows = n_g - p * CR
        row0 = g * A + p * CR
        return row0, vrows

    def fire_in(t, k):
        row0, vrows = info(t)

        @pl.when((t < NCH) & (vrows > 0))
        def _():
            pltpu.make_async_copy(
                atom_hbm.at[pl.ds(row0, CR)], a_bufs[k], in_sems[k]
            ).start()
            pltpu.make_async_copy(
                res_hbm.at[pl.ds(row0, CR)], r_bufs[k], in_sems[k]
            ).start()

    def process(t, k, state):
        mask, zcount = state
        row0, vrows = info(t)
        valid = vrows > 0
        pend = (mask & (1 << k)) != 0

        @pl.when(valid)
        def _():
            pltpu.make_async_copy(
                atom_hbm.at[pl.ds(row0, CR)], a_bufs[k], in_sems[k]
            ).wait()
            pltpu.make_async_copy(
                res_hbm.at[pl.ds(row0, CR)], r_bufs[k], in_sems[k]
            ).wait()

            @pl.when(pend)
            def _():
                pltpu.make_async_copy(
                    o_bufs[k], out_hbm.at[pl.ds(0, CR)], out_sems[k]
                ).wait()

            @pl.when(vrows >= CR)
            def _():
                def full_body(r, c2):
                    for c in range(D // 16):
                        sl = pl.ds(c * 16, 16)
                        o_bufs[k][r, sl] = a_bufs[k][r, sl] + r_bufs[k][r, sl]
                    return c2

                lax.fori_loop(0, CR, full_body, 0)

            @pl.when(vrows < CR)
            def _():
                def part_body(r, c2):
                    keep = r < vrows
                    for c in range(D // 16):
                        sl = pl.ds(c * 16, 16)
                        s = a_bufs[k][r, sl] + r_bufs[k][r, sl]
                        o_bufs[k][r, sl] = jnp.where(keep, s, zero16)
                    return c2

                lax.fori_loop(0, CR, part_body, 0)

            pltpu.make_async_copy(
                o_bufs[k], out_hbm.at[pl.ds(row0, CR)], out_sems[k]
            ).start()

        @pl.when(jnp.logical_not(valid))
        def _():
            pltpu.make_async_copy(
                z_buf, out_hbm.at[pl.ds(row0, CR)], zero_sem
            ).start()

        mask = jnp.where(valid, mask | (1 << k), mask).astype(jnp.int32)
        zcount = jnp.where(valid, zcount, zcount + 1).astype(jnp.int32)
        return mask, zcount

    fire_in(jnp.int32(0), 0)
    fire_in(jnp.int32(1), 1)
    fire_in(jnp.int32(2), 2)

    def group_body(grp, state):
        mask, zcount = state
        for kk in range(NSLOT):
            t = grp * NSLOT + kk
            fire_in(t + 3, (kk + 3) % NSLOT)
            mask, zcount = process(t, kk, (mask, zcount))
        return mask, zcount

    mask, zcount = lax.fori_loop(
        0, NCH // NSLOT, group_body, (jnp.int32(0), jnp.int32(0))
    )

    for k in range(NSLOT):
        @pl.when((mask & (1 << k)) != 0)
        def _():
            pltpu.make_async_copy(
                o_bufs[k], out_hbm.at[pl.ds(0, CR)], out_sems[k]
            ).wait()

    def zdrain(i, carry):
        pltpu.make_async_copy(
            z_buf, out_hbm.at[pl.ds(0, CR)], zero_sem
        ).wait()
        return carry

    lax.fori_loop(0, zcount, zdrain, 0)


def kernel(atom_features, mol_slice, residual_features):
    n = jnp.pad(mol_slice[:, 0].astype(jnp.int32), (0, 16))
    out = _block_end_sc(
        atom_features.reshape(B * A, D), n, residual_features.reshape(B * A, D)
    )
    return out.reshape(B, A, D)


# speedup vs baseline: 13.0591x; 1.0069x over previous
import functools

import jax
import jax.numpy as jnp
from jax import lax
from jax.experimental import pallas as pl
from jax.experimental.pallas import tpu as pltpu
from jax.experimental.pallas import tpu_sc as plsc

B, A, D = 16, 2048, 256
CR = 32
POS = A // CR
NC, NS = 2, 16
NW = NC * NS
NCH = 2 * B
NSLOT = 4

_mesh = plsc.VectorSubcoreMesh(core_axis_name="c", subcore_axis_name="s")


@functools.partial(
    pl.kernel,
    mesh=_mesh,
    out_type=jax.ShapeDtypeStruct((B * A, D), jnp.float32),
    scratch_types=(
        [pltpu.VMEM((32,), jnp.int32)]
        + [pltpu.VMEM((CR, D), jnp.float32) for _ in range(3 * NSLOT + 1)]
        + [pltpu.SemaphoreType.DMA for _ in range(2 * NSLOT + 1)]
    ),
)
def _block_end_sc(atom_hbm, n_hbm, res_hbm, out_hbm, n_v, *bufs):
    a_bufs = bufs[0:NSLOT]
    r_bufs = bufs[NSLOT:2 * NSLOT]
    o_bufs = bufs[2 * NSLOT:3 * NSLOT]
    z_buf = bufs[3 * NSLOT]
    in_sems = bufs[3 * NSLOT + 1: 3 * NSLOT + 1 + NSLOT]
    out_sems = bufs[3 * NSLOT + 1 + NSLOT: 3 * NSLOT + 1 + 2 * NSLOT]
    zero_sem = bufs[3 * NSLOT + 1 + 2 * NSLOT]

    w = lax.axis_index("s") * NC + lax.axis_index("c")
    pltpu.sync_copy(n_hbm, n_v)

    zero16 = jnp.zeros((16,), jnp.float32)

    def zbody(r, carry):
        for c in range(D // 16):
            z_buf[r, pl.ds(c * 16, 16)] = zero16
        return carry

    lax.fori_loop(0, CR, zbody, 0)

    def info(t):
        g = jnp.minimum(t // 2, B - 1)
        pe = (2 * (w + g)) % POS
        p = jnp.where(t % 2 == 0, pe, POS - 1 - pe)
        n_g = n_v[pl.ds(g, 16)][0]
        vrows = n_g - p * CR
        row0 = g * A + p * CR
        return row0, vrows

    def fire_in(t, k):
        row0, vrows = info(t)

        @pl.when((t < NCH) & (vrows > 0))
        def _():
            pltpu.make_async_copy(
                atom_hbm.at[pl.ds(row0, CR)], a_bufs[k], in_sems[k]
            ).start()
            pltpu.make_async_copy(
                res_hbm.at[pl.ds(row0, CR)], r_bufs[k], in_sems[k]
            ).start()

    def process(t, k, state):
        mask, zcount = state
        row0, vrows = info(t)
        valid = vrows > 0
        pend = (mask & (1 << k)) != 0

        @pl.when(valid)
        def _():
            pltpu.make_async_copy(
                atom_hbm.at[pl.ds(row0, CR)], a_bufs[k], in_sems[k]
            ).wait()
            pltpu.make_async_copy(
                res_hbm.at[pl.ds(row0, CR)], r_bufs[k], in_sems[k]
            ).wait()

            @pl.when(pend)
            def _():
                pltpu.make_async_copy(
                    o_bufs[k], out_hbm.at[pl.ds(0, CR)], out_sems[k]
                ).wait()

            @pl.when(vrows >= CR)
            def _():
                def full_body(r, c2):
                    for c in range(D // 16):
                        sl = pl.ds(c * 16, 16)
                        o_bufs[k][r, sl] = a_bufs[k][r, sl] + r_bufs[k][r, sl]
                    return c2

                lax.fori_loop(0, CR, full_body, 0)

            @pl.when(vrows < CR)
            def _():
                def part_body(r, c2):
                    keep = r < vrows
                    for c in range(D // 16):
                        sl = pl.ds(c * 16, 16)
                        s = a_bufs[k][r, sl] + r_bufs[k][r, sl]
                        o_bufs[k][r, sl] = jnp.where(keep, s, zero16)
                    return c2

                lax.fori_loop(0, CR, part_body, 0)

            pltpu.make_async_copy(
                o_bufs[k], out_hbm.at[pl.ds(row0, CR)], out_sems[k]
            ).start()

        @pl.when(jnp.logical_not(valid))
        def _():
            pltpu.make_async_copy(
                z_buf, out_hbm.at[pl.ds(row0, CR)], zero_sem
            ).start()

        mask = jnp.where(valid, mask | (1 << k), mask).astype(jnp.int32)
        zcount = jnp.where(valid, zcount, zcount + 1).astype(jnp.int32)
        return mask, zcount

    fire_in(jnp.int32(0), 0)
    fire_in(jnp.int32(1), 1)
    fire_in(jnp.int32(2), 2)

    def group_body(grp, state):
        mask, zcount = state
        for kk in range(NSLOT):
            t = grp * NSLOT + kk
            fire_in(t + 3, (kk + 3) % NSLOT)
            mask, zcount = process(t, kk, (mask, zcount))
        return mask, zcount

    mask, zcount = lax.fori_loop(
        0, NCH // NSLOT, group_body, (jnp.int32(0), jnp.int32(0))
    )

    for k in range(NSLOT):
        @pl.when((mask & (1 << k)) != 0)
        def _():
            pltpu.make_async_copy(
                o_bufs[k], out_hbm.at[pl.ds(0, CR)], out_sems[k]
            ).wait()

    def zdrain(i, carry):
        pltpu.make_async_copy(
            z_buf, out_hbm.at[pl.ds(0, CR)], zero_sem
        ).wait()
        return carry

    lax.fori_loop(0, zcount, zdrain, 0)


def kernel(atom_features, mol_slice, residual_features):
    n = jnp.pad(mol_slice[:, 0].astype(jnp.int32), (0, 16))
    out = _block_end_sc(
        atom_features.reshape(B * A, D), n, residual_features.reshape(B * A, D)
    )
    return out.reshape(B, A, D)
